# baseline (speedup 1.0000x reference)
import functools
import math

import jax
import jax.numpy as jnp
from jax.experimental import pallas as pl
from jax.experimental.pallas import tpu as pltpu
from jax.experimental.pallas import tpu_sc as plsc

POOL = 64
LENGTH = 16
D = 768
TOPK = 8
B = 4
S = 2048
KD = LENGTH * D
NM = 3 * POOL

NWORK = 32
RPW = B * S // NWORK
RCH = 64
NCH = RPW // RCH
NLG = D // 16

NSTEP = 4
KCHUNK = KD // NSTEP

_HALF_PI = math.pi / 2.0


def _sc_mean_body(x_hbm, out_hbm, buf, accv, sem0, sem1):
    c = jax.lax.axis_index("c")
    s = jax.lax.axis_index("s")
    wid = s * 2 + c
    base = wid * RPW
    sems = [sem0, sem1]
    cps = []
    cp0 = pltpu.make_async_copy(x_hbm.at[pl.ds(base, RCH)], buf.at[0], sems[0])
    cp0.start()
    cps.append(cp0)
    accs = tuple(jnp.zeros((16,), jnp.float32) for _ in range(NLG))
    for ch in range(NCH):
        if ch + 1 < NCH:
            nb = (ch + 1) % 2
            cpn = pltpu.make_async_copy(
                x_hbm.at[pl.ds(base + (ch + 1) * RCH, RCH)], buf.at[nb], sems[nb])
            cpn.start()
            cps.append(cpn)
        cps[ch].wait()

        def row_body(r, a, _b=ch % 2):
            return tuple(a[j] + buf[_b, r, pl.ds(16 * j, 16)] for j in range(NLG))

        accs = jax.lax.fori_loop(0, RCH, row_body, accs)
    for j in range(NLG):
        accv[0, pl.ds(16 * j, 16)] = accs[j]
    pltpu.sync_copy(accv, out_hbm.at[pl.ds(wid, 1)])


@functools.partial(
    pl.kernel,
    out_type=jax.ShapeDtypeStruct((NWORK, D), jnp.float32),
    mesh=plsc.VectorSubcoreMesh(core_axis_name="c", subcore_axis_name="s"),
    scratch_types=[
        pltpu.VMEM((2, RCH, D), jnp.float32),
        pltpu.VMEM((1, D), jnp.float32),
        pltpu.SemaphoreType.DMA,
        pltpu.SemaphoreType.DMA,
    ],
)
def _sc_mean(x_hbm, out_hbm, buf, accv, sem0, sem1):
    _sc_mean_body(x_hbm, out_hbm, buf, accv, sem0, sem1)


def _asin_poly(x):
    z = x * x
    p = jnp.float32(4.2163199048e-2)
    p = p * z + jnp.float32(2.4181311049e-2)
    p = p * z + jnp.float32(4.5470025998e-2)
    p = p * z + jnp.float32(7.4953002686e-2)
    p = p * z + jnp.float32(1.6666752422e-1)
    return x + x * z * p


def _arcsin01(x):
    s = jnp.sqrt(jnp.maximum(0.5 * (1.0 - x), 0.0))
    r_big = jnp.float32(_HALF_PI) - 2.0 * _asin_poly(s)
    return jnp.where(x > 0.5, r_big, _asin_poly(jnp.minimum(x, 0.5)))


def _l2n(v):
    ss = jnp.sum(v * v, axis=1, keepdims=True)
    return v * jax.lax.rsqrt(jnp.maximum(ss, 1e-12))


def _topk_weights(sim):
    vi = sim[:, :, None]
    vj = sim[:, None, :]
    ii = jax.lax.broadcasted_iota(jnp.int32, (B, POOL, POOL), 1)
    jj = jax.lax.broadcasted_iota(jnp.int32, (B, POOL, POOL), 2)
    beats = jnp.where((vi > vj) | ((vi == vj) & (ii < jj)), 1.0, 0.0)
    rank = jnp.sum(beats, axis=1)
    return jnp.where(rank < TOPK, jnp.float32(1.0 / TOPK), 0.0)


def _pair_ddl(blk, nrow, ncol):
    cos = blk / (nrow * ncol)
    return jnp.sum(_arcsin01(jnp.clip(cos, 0.0, 1.0)), keepdims=True)


def _cube(x):
    c = jnp.maximum(x, 1e-6)
    return c * c * c


def _gram_body(p_ref, a_ref, o_ref, g_ref, gacc):
    i = pl.program_id(0)
    mk = jnp.concatenate([p_ref[...], a_ref[...], o_ref[...]], axis=0)
    g = jax.lax.dot_general(mk, mk, (((1,), (1,)), ((), ())),
                            preferred_element_type=jnp.float32)

    @pl.when(i == 0)
    def _():
        gacc[...] = g

    @pl.when(i > 0)
    def _():
        gacc[...] += g

    @pl.when(i == NSTEP - 1)
    def _():
        g_ref[...] = gacc[...]


def _fin_body(xs_ref, g_ref, p_ref, o_ref, pk_ref, ok_ref,
              ddl_ref, ortho_ref, sim_ref, osim_ref, bp_ref, bo_ref,
              wp_s, wo_s):
    i = pl.program_id(0)

    @pl.when(i == 0)
    def _():
        pr = jax.lax.broadcasted_iota(jnp.int32, (B, NWORK), 0)
        pc = jax.lax.broadcasted_iota(jnp.int32, (B, NWORK), 1)
        pair = jnp.where(pc // (NWORK // B) == pr, 1.0, 0.0)
        xmean = jax.lax.dot_general(pair, xs_ref[...], (((1,), (0,)), ((), ())),
                                    preferred_element_type=jnp.float32)
        xn = _l2n(xmean * jnp.float32(1.0 / S))
        pkn = _l2n(pk_ref[...])
        okn = _l2n(ok_ref[...])
        sim = jax.lax.dot_general(xn, pkn, (((1,), (1,)), ((), ())),
                                  preferred_element_type=jnp.float32)
        osim = jax.lax.dot_general(xn, okn, (((1,), (1,)), ((), ())),
                                   preferred_element_type=jnp.float32)
        sim_ref[...] = sim
        osim_ref[...] = osim
        wp_s[...] = _topk_weights(sim)
        wo_s[...] = _topk_weights(osim)

        g_all = g_ref[...]
        r = jax.lax.broadcasted_iota(jnp.int32, (NM, NM), 0)
        c = jax.lax.broadcasted_iota(jnp.int32, (NM, NM), 1)
        eye = jnp.where(r == c, 1.0, 0.0)
        geye = g_all * eye
        nrow = jnp.maximum(jnp.sqrt(jnp.sum(geye, axis=1, keepdims=True)), 1e-8)
        ncol = jnp.maximum(jnp.sqrt(jnp.sum(geye, axis=0, keepdims=True)), 1e-8)

        pp = g_all[0:POOL, 0:POOL]
        aa = g_all[POOL:2 * POOL, POOL:2 * POOL]
        oo = g_all[2 * POOL:NM, 2 * POOL:NM]
        ap = g_all[POOL:2 * POOL, 0:POOL]
        op = g_all[2 * POOL:NM, 0:POOL]
        ao = g_all[POOL:2 * POOL, 2 * POOL:NM]

        ddl = (_pair_ddl(ap, nrow[POOL:2 * POOL], ncol[:, 0:POOL])
               + _pair_ddl(op, nrow[2 * POOL:NM], ncol[:, 0:POOL])
               + _pair_ddl(ao, nrow[POOL:2 * POOL], ncol[:, 2 * POOL:NM]))
        ddl_ref[...] = ddl * jnp.float32(2.0 / (POOL * POOL))

        eye64 = eye[0:POOL, 0:POOL]
        ortho = (jnp.sum((pp - eye64) ** 2, keepdims=True)
                 + jnp.sum((aa - eye64) ** 2, keepdims=True)
                 + jnp.sum((oo - eye64) ** 2, keepdims=True))
        ortho_ref[...] = ortho * jnp.float32(1.0 / (POOL * POOL))

    third = jnp.float32(1.0 / 3.0)
    gm = jax.lax.dot_general(wp_s[...], _cube(p_ref[...]), (((1,), (0,)), ((), ())),
                             preferred_element_type=jnp.float32)
    go = jax.lax.dot_general(wo_s[...], _cube(o_ref[...]), (((1,), (0,)), ((), ())),
                             preferred_element_type=jnp.float32)
    bp_ref[...] = jnp.exp(jnp.log(gm) * third)
    bo_ref[...] = jnp.exp(jnp.log(go) * third)


@jax.jit
def kernel(x_embed, prompt, prompt_key, attr_prompt, obj_prompt, obj_prompt_key):
    x2 = x_embed.reshape(B * S, D)
    p2 = prompt.reshape(POOL, KD)
    a2 = attr_prompt.reshape(POOL, KD)
    o2 = obj_prompt.reshape(POOL, KD)

    xsum32 = _sc_mean(x2)

    kchunk_spec = pl.BlockSpec((POOL, KCHUNK), lambda i: (0, i))
    full = lambda shape: pl.BlockSpec(shape, lambda i: (0,) * len(shape))

    g_mat = pl.pallas_call(
        _gram_body,
        grid=(NSTEP,),
        in_specs=[kchunk_spec, kchunk_spec, kchunk_spec],
        out_specs=full((NM, NM)),
        out_shape=jax.ShapeDtypeStruct((NM, NM), jnp.float32),
        scratch_shapes=[pltpu.VMEM((NM, NM), jnp.float32)],
        compiler_params=pltpu.CompilerParams(
            dimension_semantics=("arbitrary",)),
    )(p2, a2, o2)

    outs = pl.pallas_call(
        _fin_body,
        grid=(NSTEP,),
        in_specs=[
            full((NWORK, D)), full((NM, NM)),
            kchunk_spec, kchunk_spec,
            full((POOL, D)), full((POOL, D)),
        ],
        out_specs=[
            full((1, 1)), full((1, 1)),
            full((B, POOL)), full((B, POOL)),
            pl.BlockSpec((B, KCHUNK), lambda i: (0, i)),
            pl.BlockSpec((B, KCHUNK), lambda i: (0, i)),
        ],
        out_shape=[
            jax.ShapeDtypeStruct((1, 1), jnp.float32),
            jax.ShapeDtypeStruct((1, 1), jnp.float32),
            jax.ShapeDtypeStruct((B, POOL), jnp.float32),
            jax.ShapeDtypeStruct((B, POOL), jnp.float32),
            jax.ShapeDtypeStruct((B, KD), jnp.float32),
            jax.ShapeDtypeStruct((B, KD), jnp.float32),
        ],
        scratch_shapes=[
            pltpu.VMEM((B, POOL), jnp.float32),
            pltpu.VMEM((B, POOL), jnp.float32),
        ],
        compiler_params=pltpu.CompilerParams(
            dimension_semantics=("arbitrary",)),
    )(xsum32, g_mat, p2, o2, prompt_key, obj_prompt_key)

    ddl, ortho, sim, osim, bp, bo = outs
    return (ddl[0, 0], ortho[0, 0], sim, osim,
            bp.reshape(B, LENGTH, D), bo.reshape(B, LENGTH, D))


# speedup vs baseline: 2.1971x; 2.1971x over previous
import math

import jax
import jax.numpy as jnp
from jax.experimental import pallas as pl
from jax.experimental.pallas import tpu as pltpu

POOL = 64
LENGTH = 16
D = 768
TOPK = 8
B = 4
S = 2048
NM = 3 * POOL

NSTEP = 4
XROWS = B * S // NSTEP
LCH = 8

_HALF_PI = math.pi / 2.0


def _asin_poly(x):
    z = x * x
    p = jnp.float32(4.2163199048e-2)
    p = p * z + jnp.float32(2.4181311049e-2)
    p = p * z + jnp.float32(4.5470025998e-2)
    p = p * z + jnp.float32(7.4953002686e-2)
    p = p * z + jnp.float32(1.6666752422e-1)
    return x + x * z * p


def _arcsin01(x):
    s = jnp.sqrt(jnp.maximum(0.5 * (1.0 - x), 0.0))
    r_big = jnp.float32(_HALF_PI) - 2.0 * _asin_poly(s)
    return jnp.where(x > 0.5, r_big, _asin_poly(jnp.minimum(x, 0.5)))


def _l2n(v):
    ss = jnp.sum(v * v, axis=1, keepdims=True)
    return v * jax.lax.rsqrt(jnp.maximum(ss, 1e-12))


def _topk_weights(sim):
    vi = sim[:, :, None]
    vj = sim[:, None, :]
    ii = jax.lax.broadcasted_iota(jnp.int32, (B, POOL, POOL), 1)
    jj = jax.lax.broadcasted_iota(jnp.int32, (B, POOL, POOL), 2)
    beats = jnp.where((vi > vj) | ((vi == vj) & (ii < jj)), 1.0, 0.0)
    rank = jnp.sum(beats, axis=1)
    return jnp.where(rank < TOPK, jnp.float32(1.0 / TOPK), 0.0)


def _pair_ddl(blk, nrow, ncol):
    cos = blk / (nrow * ncol)
    return jnp.sum(_arcsin01(jnp.clip(cos, 0.0, 1.0)), keepdims=True)


def _cube(x):
    c = jnp.maximum(x, 1e-6)
    return c * c * c


def _body(x_ref, p_ref, a_ref, o_ref, pk_ref, ok_ref,
          ddl_ref, ortho_ref, sim_ref, osim_ref, bp_ref, bo_ref,
          xsum, gacc, pcube, ocube):
    i = pl.program_id(0)

    xsum[pl.ds(i, 1), :] = jnp.sum(x_ref[...], axis=0, keepdims=True)

    @pl.when(i % 2 == 0)
    def _():
        pk3 = p_ref[...]
        ak3 = a_ref[...]
        ok3 = o_ref[...]
        pcube[:, pl.ds((i // 2) * LCH, LCH), :] = _cube(pk3)
        ocube[:, pl.ds((i // 2) * LCH, LCH), :] = _cube(ok3)

        g = None
        for j in range(LCH):
            mk = jnp.concatenate([pk3[:, j], ak3[:, j], ok3[:, j]], axis=0)
            gj = jax.lax.dot_general(mk, mk, (((1,), (1,)), ((), ())),
                                     preferred_element_type=jnp.float32)
            g = gj if g is None else g + gj

        @pl.when(i == 0)
        def _():
            gacc[...] = g

        @pl.when(i > 0)
        def _():
            gacc[...] += g

    @pl.when(i == NSTEP - 1)
    def _():
        pr = jax.lax.broadcasted_iota(jnp.int32, (B, NSTEP), 0)
        pc = jax.lax.broadcasted_iota(jnp.int32, (B, NSTEP), 1)
        pair = jnp.where(pc // (NSTEP // B) == pr, 1.0, 0.0)
        xmean = jax.lax.dot_general(pair, xsum[...], (((1,), (0,)), ((), ())),
                                    preferred_element_type=jnp.float32)
        xn = _l2n(xmean * jnp.float32(1.0 / S))
        pkn = _l2n(pk_ref[...])
        okn = _l2n(ok_ref[...])
        sim = jax.lax.dot_general(xn, pkn, (((1,), (1,)), ((), ())),
                                  preferred_element_type=jnp.float32)
        osim = jax.lax.dot_general(xn, okn, (((1,), (1,)), ((), ())),
                                   preferred_element_type=jnp.float32)
        sim_ref[...] = sim
        osim_ref[...] = osim

        wp = _topk_weights(sim)
        wo = _topk_weights(osim)
        third = jnp.float32(1.0 / 3.0)
        for j in range(LENGTH):
            gm = jax.lax.dot_general(wp, pcube[:, j, :], (((1,), (0,)), ((), ())),
                                     preferred_element_type=jnp.float32)
            go = jax.lax.dot_general(wo, ocube[:, j, :], (((1,), (0,)), ((), ())),
                                     preferred_element_type=jnp.float32)
            bp_ref[:, pl.ds(j, 1), :] = jnp.exp(jnp.log(gm) * third)[:, None, :]
            bo_ref[:, pl.ds(j, 1), :] = jnp.exp(jnp.log(go) * third)[:, None, :]

        g_all = gacc[...]
        r = jax.lax.broadcasted_iota(jnp.int32, (NM, NM), 0)
        c = jax.lax.broadcasted_iota(jnp.int32, (NM, NM), 1)
        eye = jnp.where(r == c, 1.0, 0.0)
        geye = g_all * eye
        nrow = jnp.maximum(jnp.sqrt(jnp.sum(geye, axis=1, keepdims=True)), 1e-8)
        ncol = jnp.maximum(jnp.sqrt(jnp.sum(geye, axis=0, keepdims=True)), 1e-8)

        pp = g_all[0:POOL, 0:POOL]
        aa = g_all[POOL:2 * POOL, POOL:2 * POOL]
        oo = g_all[2 * POOL:NM, 2 * POOL:NM]
        ap = g_all[POOL:2 * POOL, 0:POOL]
        op = g_all[2 * POOL:NM, 0:POOL]
        ao = g_all[POOL:2 * POOL, 2 * POOL:NM]

        ddl = (_pair_ddl(ap, nrow[POOL:2 * POOL], ncol[:, 0:POOL])
               + _pair_ddl(op, nrow[2 * POOL:NM], ncol[:, 0:POOL])
               + _pair_ddl(ao, nrow[POOL:2 * POOL], ncol[:, 2 * POOL:NM]))
        ddl_ref[...] = ddl * jnp.float32(2.0 / (POOL * POOL))

        eye64 = eye[0:POOL, 0:POOL]
        ortho = (jnp.sum((pp - eye64) ** 2, keepdims=True)
                 + jnp.sum((aa - eye64) ** 2, keepdims=True)
                 + jnp.sum((oo - eye64) ** 2, keepdims=True))
        ortho_ref[...] = ortho * jnp.float32(1.0 / (POOL * POOL))


@jax.jit
def kernel(x_embed, prompt, prompt_key, attr_prompt, obj_prompt, obj_prompt_key):
    x2 = x_embed.reshape(B * S, D)

    lchunk = pl.BlockSpec((POOL, LCH, D), lambda i: (0, i // 2, 0))
    full = lambda shape: pl.BlockSpec(shape, lambda i: (0,) * len(shape))
    outs = pl.pallas_call(
        _body,
        grid=(NSTEP,),
        in_specs=[
            pl.BlockSpec((XROWS, D), lambda i: (i, 0)),
            lchunk, lchunk, lchunk,
            full((POOL, D)), full((POOL, D)),
        ],
        out_specs=[
            full((1, 1)), full((1, 1)),
            full((B, POOL)), full((B, POOL)),
            full((B, LENGTH, D)), full((B, LENGTH, D)),
        ],
        out_shape=[
            jax.ShapeDtypeStruct((1, 1), jnp.float32),
            jax.ShapeDtypeStruct((1, 1), jnp.float32),
            jax.ShapeDtypeStruct((B, POOL), jnp.float32),
            jax.ShapeDtypeStruct((B, POOL), jnp.float32),
            jax.ShapeDtypeStruct((B, LENGTH, D), jnp.float32),
            jax.ShapeDtypeStruct((B, LENGTH, D), jnp.float32),
        ],
        scratch_shapes=[
            pltpu.VMEM((NSTEP, D), jnp.float32),
            pltpu.VMEM((NM, NM), jnp.float32),
            pltpu.VMEM((POOL, LENGTH, D), jnp.float32),
            pltpu.VMEM((POOL, LENGTH, D), jnp.float32),
        ],
        compiler_params=pltpu.CompilerParams(
            dimension_semantics=("arbitrary",)),
    )(x2, prompt, attr_prompt, obj_prompt, prompt_key, obj_prompt_key)

    ddl, ortho, sim, osim, bp, bo = outs
    return (ddl[0, 0], ortho[0, 0], sim, osim, bp, bo)
